# speedup vs baseline: 5.5032x; 5.5032x over previous
import functools

import jax
import jax.numpy as jnp
from jax import lax
from jax.experimental import pallas as pl
from jax.experimental.pallas import tpu as pltpu
from jax.experimental.pallas import tpu_sc as plsc

NC = 2
NS = 16
NW = NC * NS
L = 16

B, SEQ = 16384, 200
N_IDS = B * SEQ
IDS_PER_TILE = N_IDS // NW
CHUNK = 12800
N_CHUNKS = IDS_PER_TILE // CHUNK
VECS = CHUNK // L


def _gather(arr, idx):
    return arr.at[idx].get(mode="promise_in_bounds")


@jax.jit
def _sc_lookup(ids_flat, params):
    mesh = plsc.VectorSubcoreMesh(core_axis_name="c", subcore_axis_name="s")

    @functools.partial(
        pl.kernel,
        mesh=mesh,
        out_type=jax.ShapeDtypeStruct((2 * N_IDS,), jnp.float32),
        scratch_types=[
            pltpu.VMEM((CHUNK,), jnp.int32),
            pltpu.VMEM((2 * CHUNK,), jnp.float32),
            pltpu.VMEM((48,), jnp.float32),
        ],
    )
    def run(ids_hbm, params_hbm, out_hbm, ids_v, out_v, par_v):
        wid = lax.axis_index("s") * NC + lax.axis_index("c")
        pltpu.sync_copy(params_hbm, par_v)

        iota = lax.iota(jnp.int32, L)
        e0 = par_v[pl.ds(0, L)]
        e1 = par_v[pl.ds(16, L)]
        wb = par_v[pl.ds(32, L)]
        v = iota & 7
        p = iota >> 3
        acc = _gather(wb, 8 + p)
        for d in range(4):
            eidx = v * 4 + d
            e_lo = _gather(e0, eidx & 15)
            e_hi = _gather(e1, eidx & 15)
            e = jnp.where(eidx < 16, e_lo, e_hi)
            w = _gather(wb, p * 4 + d)
            acc = acc + e * w
        t = acc

        ilo = iota >> 1
        ihi = 8 + (iota >> 1)
        paroff = (iota & 1) * 8

        base = wid * IDS_PER_TILE

        @pl.loop(0, N_CHUNKS)
        def _(c):
            off = base + c * CHUNK
            pltpu.sync_copy(ids_hbm.at[pl.ds(off, CHUNK)], ids_v)

            @pl.loop(0, VECS)
            def _(k):
                idv = ids_v[pl.ds(k * L, L)]
                lo = _gather(idv, ilo)
                hi = _gather(idv, ihi)
                out_v[pl.ds(k * 2 * L, L)] = _gather(t, lo + paroff)
                out_v[pl.ds(k * 2 * L + L, L)] = _gather(t, hi + paroff)

            pltpu.sync_copy(out_v, out_hbm.at[pl.ds(2 * off, 2 * CHUNK)])

    return run(ids_flat, params)


def kernel(input_ids, embed_weight, lm_head_weight, lm_head_bias):
    ids_flat = input_ids.reshape(-1).astype(jnp.int32)
    params = jnp.concatenate(
        [
            embed_weight.reshape(-1),
            lm_head_weight.reshape(-1),
            lm_head_bias,
            jnp.zeros((6,), jnp.float32),
        ]
    )
    out_flat = _sc_lookup(ids_flat, params)
    return out_flat.reshape(B, SEQ, 2)


# speedup vs baseline: 136.9223x; 24.8804x over previous
import functools

import jax
import jax.numpy as jnp
from jax import lax
from jax.experimental import pallas as pl
from jax.experimental.pallas import tpu as pltpu
from jax.experimental.pallas import tpu_sc as plsc

NC = 2
NS = 16
NW = NC * NS
L = 16

B, SEQ = 16384, 200
N_IDS = B * SEQ
IDS_PER_TILE = N_IDS // NW
CHUNK = 12800
N_CHUNKS = IDS_PER_TILE // CHUNK
GROUPS = CHUNK // 128


def _gather(arr, idx):
    return arr.at[idx].get(mode="promise_in_bounds")


@jax.jit
def _sc_lookup(ids_flat, params):
    mesh = plsc.VectorSubcoreMesh(core_axis_name="c", subcore_axis_name="s")

    @functools.partial(
        pl.kernel,
        mesh=mesh,
        out_type=jax.ShapeDtypeStruct((2 * N_IDS,), jnp.float32),
        scratch_types=[
            pltpu.VMEM((CHUNK,), jnp.int32),
            pltpu.VMEM((2 * CHUNK,), jnp.float32),
            pltpu.VMEM((48,), jnp.float32),
        ],
    )
    def run(ids_hbm, params_hbm, out_hbm, ids_v, out_v, par_v):
        wid = lax.axis_index("s") * NC + lax.axis_index("c")
        pltpu.sync_copy(params_hbm, par_v)

        iota = lax.iota(jnp.int32, L)
        e0 = par_v[pl.ds(0, L)]
        e1 = par_v[pl.ds(16, L)]
        wb = par_v[pl.ds(32, L)]
        v = iota & 7
        p = iota >> 3
        acc = _gather(wb, 8 + p)
        for d in range(4):
            eidx = v * 4 + d
            e_lo = _gather(e0, eidx & 15)
            e_hi = _gather(e1, eidx & 15)
            e = jnp.where(eidx < 16, e_lo, e_hi)
            w = _gather(wb, p * 4 + d)
            acc = acc + e * w
        t = acc

        base = wid * IDS_PER_TILE

        @pl.loop(0, N_CHUNKS)
        def _(c):
            off = base + c * CHUNK
            pltpu.sync_copy(ids_hbm.at[pl.ds(off, CHUNK)], ids_v)

            @pl.loop(0, GROUPS)
            def _(g):
                for u in range(8):
                    idv = ids_v[pl.ds(g * 128 + u * L, L)]
                    out_v[pl.ds(g * 256 + u * L, L)] = _gather(t, idv)
                    out_v[pl.ds(g * 256 + 128 + u * L, L)] = _gather(t, idv + 8)

            pltpu.sync_copy(out_v, out_hbm.at[pl.ds(2 * off, 2 * CHUNK)])

    return run(ids_flat, params)


def kernel(input_ids, embed_weight, lm_head_weight, lm_head_bias):
    ids_flat = input_ids.T.reshape(-1).astype(jnp.int32)
    params = jnp.concatenate(
        [
            embed_weight.reshape(-1),
            lm_head_weight.reshape(-1),
            lm_head_bias,
            jnp.zeros((6,), jnp.float32),
        ]
    )
    out_flat = _sc_lookup(ids_flat, params)
    return (
        out_flat.reshape(SEQ, B // 128, 2, 128)
        .transpose(1, 3, 0, 2)
        .reshape(B, SEQ, 2)
    )


# speedup vs baseline: 159.6313x; 1.1659x over previous
import functools

import jax
import jax.numpy as jnp
from jax import lax
from jax.experimental import pallas as pl
from jax.experimental.pallas import tpu as pltpu
from jax.experimental.pallas import tpu_sc as plsc

NC = 2
NS = 16
NW = NC * NS
L = 16

B, SEQ = 16384, 200
N_IDS = B * SEQ
IDS_PER_TILE = N_IDS // NW
CHUNK = 12800
N_CHUNKS = IDS_PER_TILE // CHUNK
GROUPS = CHUNK // 128


def _gather(arr, idx):
    return arr.at[idx].get(mode="promise_in_bounds")


@jax.jit
def _sc_lookup(ids_flat, params):
    mesh = plsc.VectorSubcoreMesh(core_axis_name="c", subcore_axis_name="s")

    @functools.partial(
        pl.kernel,
        mesh=mesh,
        out_type=jax.ShapeDtypeStruct((2 * N_IDS,), jnp.float32),
        scratch_types=[
            pltpu.VMEM((CHUNK,), jnp.int32),
            pltpu.VMEM((CHUNK,), jnp.int32),
            pltpu.VMEM((2 * CHUNK,), jnp.float32),
            pltpu.VMEM((2 * CHUNK,), jnp.float32),
            pltpu.VMEM((48,), jnp.float32),
            pltpu.SemaphoreType.DMA,
            pltpu.SemaphoreType.DMA,
            pltpu.SemaphoreType.DMA,
            pltpu.SemaphoreType.DMA,
        ],
    )
    def run(ids_hbm, params_hbm, out_hbm, ids_v0, ids_v1, out_v0, out_v1,
            par_v, si0, si1, so0, so1):
        wid = lax.axis_index("s") * NC + lax.axis_index("c")
        base = wid * IDS_PER_TILE
        ids_bufs, out_bufs = [ids_v0, ids_v1], [out_v0, out_v1]
        isems, osems = [si0, si1], [so0, so1]
        in_handles = [
            pltpu.async_copy(ids_hbm.at[pl.ds(base + c * CHUNK, CHUNK)],
                             ids_bufs[c], isems[c])
            for c in range(2)
        ]
        pltpu.sync_copy(params_hbm, par_v)

        iota = lax.iota(jnp.int32, L)
        e0 = par_v[pl.ds(0, L)]
        e1 = par_v[pl.ds(16, L)]
        wb = par_v[pl.ds(32, L)]
        v = iota & 7
        p = iota >> 3
        acc = _gather(wb, 8 + p)
        for d in range(4):
            eidx = v * 4 + d
            e_lo = _gather(e0, eidx & 15)
            e_hi = _gather(e1, eidx & 15)
            e = jnp.where(eidx < 16, e_lo, e_hi)
            w = _gather(wb, p * 4 + d)
            acc = acc + e * w
        t = acc

        out_handles = [None, None]
        for c in range(N_CHUNKS):
            bsel = c % 2
            ids_v, out_v = ids_bufs[bsel], out_bufs[bsel]
            in_handles[bsel].wait()
            if out_handles[bsel] is not None:
                out_handles[bsel].wait()

            @pl.loop(0, GROUPS)
            def _(g, ids_v=ids_v, out_v=out_v):
                for u in range(8):
                    idv = ids_v[pl.ds(g * 128 + u * L, L)]
                    out_v[pl.ds(g * 256 + u * L, L)] = _gather(t, idv)
                    out_v[pl.ds(g * 256 + 128 + u * L, L)] = _gather(t, idv + 8)

            out_handles[bsel] = pltpu.async_copy(
                out_v, out_hbm.at[pl.ds(2 * (base + c * CHUNK), 2 * CHUNK)],
                osems[bsel])
            if c + 2 < N_CHUNKS:
                in_handles[bsel] = pltpu.async_copy(
                    ids_hbm.at[pl.ds(base + (c + 2) * CHUNK, CHUNK)],
                    ids_v, isems[bsel])
        out_handles[0].wait()
        out_handles[1].wait()

    return run(ids_flat, params)


def kernel(input_ids, embed_weight, lm_head_weight, lm_head_bias):
    ids_flat = input_ids.T.reshape(-1).astype(jnp.int32)
    params = jnp.concatenate(
        [
            embed_weight.reshape(-1),
            lm_head_weight.reshape(-1),
            lm_head_bias,
            jnp.zeros((6,), jnp.float32),
        ]
    )
    out_flat = _sc_lookup(ids_flat, params)
    return (
        out_flat.reshape(SEQ, B // 128, 2, 128)
        .transpose(1, 3, 0, 2)
        .reshape(B, SEQ, 2)
    )


# speedup vs baseline: 248.9641x; 1.5596x over previous
import functools

import jax
import jax.numpy as jnp
from jax import lax
from jax.experimental import pallas as pl
from jax.experimental.pallas import tpu as pltpu
from jax.experimental.pallas import tpu_sc as plsc

NC = 2
NS = 16
NW = NC * NS
L = 16

B, SEQ = 16384, 200
N_IDS = B * SEQ
IDS_PER_TILE = N_IDS // NW
CHUNK = 12800
N_CHUNKS = IDS_PER_TILE // CHUNK
GROUPS = CHUNK // 128


def _gather(arr, idx):
    return arr.at[idx].get(mode="promise_in_bounds")


@jax.jit
def _sc_lookup(ids_flat, params):
    mesh = plsc.VectorSubcoreMesh(core_axis_name="c", subcore_axis_name="s")

    @functools.partial(
        pl.kernel,
        mesh=mesh,
        out_type=jax.ShapeDtypeStruct((2 * N_IDS,), jnp.float32),
        scratch_types=[
            pltpu.VMEM((CHUNK,), jnp.int32),
            pltpu.VMEM((CHUNK,), jnp.int32),
            pltpu.VMEM((2 * CHUNK,), jnp.float32),
            pltpu.VMEM((2 * CHUNK,), jnp.float32),
            pltpu.VMEM((48,), jnp.float32),
            pltpu.SemaphoreType.DMA,
            pltpu.SemaphoreType.DMA,
            pltpu.SemaphoreType.DMA,
            pltpu.SemaphoreType.DMA,
        ],
    )
    def run(ids_hbm, params_hbm, out_hbm, ids_v0, ids_v1, out_v0, out_v1,
            par_v, si0, si1, so0, so1):
        wid = lax.axis_index("s") * NC + lax.axis_index("c")
        base = wid * IDS_PER_TILE
        ids_bufs, out_bufs = [ids_v0, ids_v1], [out_v0, out_v1]
        isems, osems = [si0, si1], [so0, so1]
        in_handles = [
            pltpu.async_copy(ids_hbm.at[pl.ds(base + c * CHUNK, CHUNK)],
                             ids_bufs[c], isems[c])
            for c in range(2)
        ]
        pltpu.sync_copy(params_hbm, par_v)

        iota = lax.iota(jnp.int32, L)
        e0 = par_v[pl.ds(0, L)]
        e1 = par_v[pl.ds(16, L)]
        wb = par_v[pl.ds(32, L)]
        v = iota & 7
        p = iota >> 3
        acc = _gather(wb, 8 + p)
        for d in range(4):
            eidx = v * 4 + d
            e_lo = _gather(e0, eidx & 15)
            e_hi = _gather(e1, eidx & 15)
            e = jnp.where(eidx < 16, e_lo, e_hi)
            w = _gather(wb, p * 4 + d)
            acc = acc + e * w
        t0 = acc
        t1 = _gather(acc, (iota & 7) + 8)

        out_handles = [None, None]
        for c in range(N_CHUNKS):
            bsel = c % 2
            ids_v, out_v = ids_bufs[bsel], out_bufs[bsel]
            in_handles[bsel].wait()
            if out_handles[bsel] is not None:
                out_handles[bsel].wait()

            @pl.loop(0, GROUPS)
            def _(g, ids_v=ids_v, out_v=out_v):
                idvs = [ids_v[pl.ds(g * 128 + u * L, L)] for u in range(8)]
                for u in range(8):
                    out_v[pl.ds(g * 256 + u * L, L)] = _gather(t0, idvs[u])
                for u in range(8):
                    out_v[pl.ds(g * 256 + 128 + u * L, L)] = _gather(t1, idvs[u])

            out_handles[bsel] = pltpu.async_copy(
                out_v, out_hbm.at[pl.ds(2 * (base + c * CHUNK), 2 * CHUNK)],
                osems[bsel])
            if c + 2 < N_CHUNKS:
                in_handles[bsel] = pltpu.async_copy(
                    ids_hbm.at[pl.ds(base + (c + 2) * CHUNK, CHUNK)],
                    ids_v, isems[bsel])
        out_handles[0].wait()
        out_handles[1].wait()

    return run(ids_flat, params)


def kernel(input_ids, embed_weight, lm_head_weight, lm_head_bias):
    ids_flat = input_ids.T.reshape(-1).astype(jnp.int32)
    params = jnp.concatenate(
        [
            embed_weight.reshape(-1),
            lm_head_weight.reshape(-1),
            lm_head_bias,
            jnp.zeros((6,), jnp.float32),
        ]
    )
    out_flat = _sc_lookup(ids_flat, params)
    return (
        out_flat.reshape(SEQ, B // 128, 2, 128)
        .transpose(1, 3, 0, 2)
        .reshape(B, SEQ, 2)
    )


# speedup vs baseline: 308.7846x; 1.2403x over previous
import functools

import jax
import jax.numpy as jnp
from jax import lax
from jax.experimental import pallas as pl
from jax.experimental.pallas import tpu as pltpu
from jax.experimental.pallas import tpu_sc as plsc

NC = 2
NS = 16
NW = NC * NS
L = 16

B, SEQ = 16384, 200
N_IDS = B * SEQ
JT, IT, SUB, LANE = 25, 128, 8, 128
KIT = 16
CHUNK = KIT * SUB * LANE
SEG = KIT * 2 * LANE
N_CHUNKS = N_IDS // CHUNK
MAX_SLOTS = 8
OUT_JT = SUB * IT * 2 * LANE


def _gather(arr, idx):
    return arr.at[idx].get(mode="promise_in_bounds")


@jax.jit
def _sc_lookup(ids_flat, params):
    mesh = plsc.VectorSubcoreMesh(core_axis_name="c", subcore_axis_name="s")

    @functools.partial(
        pl.kernel,
        mesh=mesh,
        out_type=jax.ShapeDtypeStruct((2 * N_IDS,), jnp.float32),
        scratch_types=[
            pltpu.VMEM((CHUNK,), jnp.int32),
            pltpu.VMEM((CHUNK,), jnp.int32),
            pltpu.VMEM((SUB * SEG,), jnp.float32),
            pltpu.VMEM((SUB * SEG,), jnp.float32),
            pltpu.VMEM((48,), jnp.float32),
            pltpu.SemaphoreType.DMA,
            pltpu.SemaphoreType.DMA,
            pltpu.SemaphoreType.DMA,
            pltpu.SemaphoreType.DMA,
        ],
    )
    def run(ids_hbm, params_hbm, out_hbm, ids_v0, ids_v1, out_v0, out_v1,
            par_v, si0, si1, so0, so1):
        wid = lax.axis_index("s") * NC + lax.axis_index("c")
        start = 6 * wid + jnp.minimum(wid, 8)
        cnt = jnp.where(wid < 8, 7, 6)
        ids_bufs, out_bufs = [ids_v0, ids_v1], [out_v0, out_v1]
        isems, osems = [si0, si1], [so0, so1]

        def in_copy(slot, issue):
            bsel = slot % 2
            cp = pltpu.make_async_copy(
                ids_hbm.at[pl.ds((start + slot) * CHUNK, CHUNK)],
                ids_bufs[bsel], isems[bsel])
            if issue:
                cp.start()
            return cp

        in_copy(0, True)
        in_copy(1, True)
        pltpu.sync_copy(params_hbm, par_v)

        iota = lax.iota(jnp.int32, L)
        e0 = par_v[pl.ds(0, L)]
        e1 = par_v[pl.ds(16, L)]
        wb = par_v[pl.ds(32, L)]
        v = iota & 7
        p = iota >> 3
        acc = _gather(wb, 8 + p)
        for d in range(4):
            eidx = v * 4 + d
            e_lo = _gather(e0, eidx & 15)
            e_hi = _gather(e1, eidx & 15)
            e = jnp.where(eidx < 16, e_lo, e_hi)
            w = _gather(wb, p * 4 + d)
            acc = acc + e * w
        t0 = acc
        t1 = _gather(acc, (iota & 7) + 8)

        def out_copies(slot, issue):
            bsel = slot % 2
            q = start + slot
            jt = q >> 3
            it0 = (q & 7) * KIT
            cps = []
            for s in range(SUB):
                cp = pltpu.make_async_copy(
                    out_bufs[bsel].at[pl.ds(s * SEG, SEG)],
                    out_hbm.at[pl.ds(jt * OUT_JT + s * (IT * 2 * LANE)
                                     + it0 * 2 * LANE, SEG)],
                    osems[bsel])
                if issue:
                    cp.start()
                cps.append(cp)
            return cps

        for slot in range(MAX_SLOTS):
            bsel = slot % 2
            ids_v, out_v = ids_bufs[bsel], out_bufs[bsel]

            @pl.when(slot < cnt)
            def _(slot=slot, bsel=bsel, ids_v=ids_v, out_v=out_v):
                in_copy(slot, False).wait()
                if slot >= 2:
                    for cp in out_copies(slot - 2, False):
                        cp.wait()

                @pl.loop(0, KIT)
                def _(n):
                    for s in range(SUB):
                        off_in = n * 1024 + s * 128
                        off_out = s * SEG + n * 256
                        idvs = [ids_v[pl.ds(off_in + u * L, L)]
                                for u in range(8)]
                        for u in range(8):
                            out_v[pl.ds(off_out + u * L, L)] = (
                                _gather(t0, idvs[u]))
                        for u in range(8):
                            out_v[pl.ds(off_out + 128 + u * L, L)] = (
                                _gather(t1, idvs[u]))

                out_copies(slot, True)
                if slot + 2 < MAX_SLOTS:
                    @pl.when(slot + 2 < cnt)
                    def _(slot=slot):
                        in_copy(slot + 2, True)

        for slot in range(MAX_SLOTS):
            @pl.when((slot < cnt) & (slot + 2 >= cnt))
            def _(slot=slot):
                for cp in out_copies(slot, False):
                    cp.wait()

    return run(ids_flat, params)


def kernel(input_ids, embed_weight, lm_head_weight, lm_head_bias):
    ids_flat = (
        input_ids.astype(jnp.int32)
        .reshape(IT, LANE, JT, SUB)
        .transpose(2, 0, 3, 1)
        .reshape(-1)
    )
    params = jnp.concatenate(
        [
            embed_weight.reshape(-1),
            lm_head_weight.reshape(-1),
            lm_head_bias,
            jnp.zeros((6,), jnp.float32),
        ]
    )
    out_flat = _sc_lookup(ids_flat, params)
    return (
        out_flat.reshape(SEQ, B // 128, 2, 128)
        .transpose(1, 3, 0, 2)
        .reshape(B, SEQ, 2)
    )
